# baseline (speedup 1.0000x reference)
import functools

import jax
import jax.numpy as jnp
from jax import lax
from jax.experimental import pallas as pl
from jax.experimental.pallas import tpu as pltpu
from jax.experimental.pallas import tpu_sc as plsc

NC, NS, LANES = 2, 16, 16
NW = NC * NS


def _silu(v):
    return v * jax.nn.sigmoid(v)


def _pack_bf16_pairs(v):
    hh = v.shape[1] // 2
    lo = v[:, :hh].astype(jnp.bfloat16).astype(jnp.float32)
    hi = v[:, hh:].astype(jnp.bfloat16).astype(jnp.float32)
    lo_u = jax.lax.bitcast_convert_type(lo, jnp.uint32) >> 16
    hi_u = jax.lax.bitcast_convert_type(hi, jnp.uint32) & jnp.uint32(0xFFFF0000)
    return jax.lax.bitcast_convert_type(lo_u | hi_u, jnp.float32)


def _unpack_bf16_pairs(p):
    u = jax.lax.bitcast_convert_type(p, jnp.uint32)
    lo = jax.lax.bitcast_convert_type(u << 16, jnp.float32)
    hi = jax.lax.bitcast_convert_type(u & jnp.uint32(0xFFFF0000), jnp.float32)
    return jnp.concatenate([lo, hi], axis=1)


def _pre_tables_body(x_ref, w1a_ref, w1b_ref, b1_ref, xa_ref, xb_ref):
    x = x_ref[...]
    xa_ref[...] = (
        jnp.dot(x, w1a_ref[...], preferred_element_type=jnp.float32) + b1_ref[...]
    )
    xb_ref[...] = jnp.dot(x, w1b_ref[...], preferred_element_type=jnp.float32)


def _pre_tables(x, w1a, w1b, b1, bn=2000):
    n, d = x.shape
    h = w1a.shape[1]
    return pl.pallas_call(
        _pre_tables_body,
        grid=(n // bn,),
        in_specs=[
            pl.BlockSpec((bn, d), lambda i: (i, 0)),
            pl.BlockSpec((d, h), lambda i: (0, 0)),
            pl.BlockSpec((d, h), lambda i: (0, 0)),
            pl.BlockSpec((1, h), lambda i: (0, 0)),
        ],
        out_specs=[
            pl.BlockSpec((bn, h), lambda i: (i, 0)),
            pl.BlockSpec((bn, h), lambda i: (i, 0)),
        ],
        out_shape=[
            jax.ShapeDtypeStruct((n, h), jnp.float32),
            jax.ShapeDtypeStruct((n, h), jnp.float32),
        ],
    )(x, w1a, w1b, b1.reshape(1, h))


def _gather_add(xa, xb, row, col, e_total, win=128):
    n, h = xa.shape
    ew = e_total // NW
    nwin = ew // win
    npair = nwin // 2
    mesh = plsc.VectorSubcoreMesh(core_axis_name="c", subcore_axis_name="s")

    @functools.partial(
        pl.kernel,
        out_type=jax.ShapeDtypeStruct((e_total, h), jnp.float32),
        mesh=mesh,
        scratch_types=[
            pltpu.VMEM((win,), jnp.int32),
            pltpu.VMEM((win,), jnp.int32),
            pltpu.VMEM((win,), jnp.int32),
            pltpu.VMEM((win,), jnp.int32),
            pltpu.VMEM((win, h), jnp.float32),
            pltpu.VMEM((win, h), jnp.float32),
            pltpu.VMEM((win, h), jnp.float32),
            pltpu.VMEM((win, h), jnp.float32),
            pltpu.SemaphoreType.DMA,
            pltpu.SemaphoreType.DMA,
            pltpu.SemaphoreType.DMA,
            pltpu.SemaphoreType.DMA,
            pltpu.SemaphoreType.DMA,
            pltpu.SemaphoreType.DMA,
        ],
    )
    def k(xa_hbm, xb_hbm, row_hbm, col_hbm, pre_hbm,
          ra0, ca0, ra1, ca1, ga0, gb0, ga1, gb1,
          si0, si1, sg0, sg1, sw0, sw1):
        wid = lax.axis_index("s") * NC + lax.axis_index("c")
        ebase = wid * ew

        def start_idx(w, ra, ca, si):
            base = pl.multiple_of(ebase + w * win, 8)
            pltpu.async_copy(row_hbm.at[pl.ds(base, win)], ra, si)
            pltpu.async_copy(col_hbm.at[pl.ds(base, win)], ca, si)

        def wait_idx(ra, ca, si):
            pltpu.make_async_copy(row_hbm.at[pl.ds(0, win)], ra, si).wait()
            pltpu.make_async_copy(col_hbm.at[pl.ds(0, win)], ca, si).wait()

        def start_gather(ra, ca, ga, gb, sg):
            pltpu.async_copy(xa_hbm.at[ra], ga, sg)
            pltpu.async_copy(xb_hbm.at[ca], gb, sg)

        def wait_gather(ra, ca, ga, gb, sg):
            pltpu.make_async_copy(xa_hbm.at[ra], ga, sg).wait()
            pltpu.make_async_copy(xb_hbm.at[ca], gb, sg).wait()

        def start_wb(w, ga, sw):
            base = pl.multiple_of(ebase + w * win, 8)
            pltpu.async_copy(ga, pre_hbm.at[pl.ds(base, win)], sw)

        def wait_wb(ga, sw):
            pltpu.make_async_copy(ga, pre_hbm.at[pl.ds(0, win)], sw).wait()

        def add_into(ga, gb):
            def add_row(r, c2):
                for j in range(h // LANES):
                    sl = pl.ds(j * LANES, LANES)
                    ga[r, sl] = ga[r, sl] + gb[r, sl]
                return c2

            lax.fori_loop(0, win, add_row, 0)

        start_idx(0, ra0, ca0, si0)
        start_idx(1, ra1, ca1, si1)
        wait_idx(ra0, ca0, si0)
        start_gather(ra0, ca0, ga0, gb0, sg0)

        def body(t, carry):
            w = 2 * t
            wait_gather(ra0, ca0, ga0, gb0, sg0)

            @pl.when(w + 2 < nwin)
            def _():
                start_idx(w + 2, ra0, ca0, si0)

            @pl.when(t > 0)
            def _():
                wait_wb(ga1, sw1)

            wait_idx(ra1, ca1, si1)
            start_gather(ra1, ca1, ga1, gb1, sg1)
            add_into(ga0, gb0)
            start_wb(w, ga0, sw0)

            wait_gather(ra1, ca1, ga1, gb1, sg1)

            @pl.when(w + 3 < nwin)
            def _():
                start_idx(w + 3, ra1, ca1, si1)

            @pl.when(w + 2 < nwin)
            def _():
                wait_wb(ga0, sw0)
                wait_idx(ra0, ca0, si0)
                start_gather(ra0, ca0, ga0, gb0, sg0)

            add_into(ga1, gb1)
            start_wb(w + 1, ga1, sw1)
            return carry

        lax.fori_loop(0, npair, body, 0)
        wait_wb(ga0, sw0)
        wait_wb(ga1, sw1)

    return k(xa, xb, row, col)


def _edge_mlp_body(nsub, pre_ref, attr_ref, w1c_ref, w2_ref, b2_ref, out_ref):
    w1c = w1c_ref[...]
    w2 = w2_ref[...]
    b2 = b2_ref[...]
    for t in range(nsub):
        a = attr_ref[0, pl.ds(t, 1), :]
        c = lax.dot_general(
            a, w1c, (((0,), (0,)), ((), ())), preferred_element_type=jnp.float32
        )
        p = pre_ref[pl.ds(t * 128, 128), :]
        h1 = _silu(p + c)
        m = _silu(jnp.dot(h1, w2, preferred_element_type=jnp.float32) + b2)
        out_ref[pl.ds(t * 128, 128), :] = jax.lax.bitcast_convert_type(
            _pack_bf16_pairs(m), jnp.uint32
        )


def _edge_mlp(pre, attr3, w1c, w2, b2, bt=1280):
    e, h = pre.shape
    nsub = bt // 128
    return pl.pallas_call(
        functools.partial(_edge_mlp_body, nsub),
        grid=(e // bt,),
        in_specs=[
            pl.BlockSpec((bt, h), lambda i: (i, 0)),
            pl.BlockSpec((1, nsub, 128), lambda i: (i, 0, 0)),
            pl.BlockSpec((1, h), lambda i: (0, 0)),
            pl.BlockSpec((h, h), lambda i: (0, 0)),
            pl.BlockSpec((1, h), lambda i: (0, 0)),
        ],
        out_specs=pl.BlockSpec((bt, h // 2), lambda i: (i, 0)),
        out_shape=jax.ShapeDtypeStruct((e, h // 2), jnp.uint32),
    )(pre, attr3, w1c, w2, b2.reshape(1, h))


def _scatter_partials(m0, r0, m1, r1, n_pad, win=64, zr=64):
    e_chunk, hp = m0.shape
    h = 2 * hp
    ept = e_chunk // NW
    nwin = ept // win
    npair = nwin // 2
    slab = n_pad // NS
    mesh = plsc.VectorSubcoreMesh(core_axis_name="c", subcore_axis_name="s")

    @functools.partial(
        pl.kernel,
        out_type=jax.ShapeDtypeStruct((NC, n_pad, h), jnp.float32),
        mesh=mesh,
        scratch_types=[
            pltpu.VMEM((win,), jnp.int32),
            pltpu.VMEM((win,), jnp.int32),
            pltpu.VMEM((win, hp), jnp.uint32),
            pltpu.VMEM((win, hp), jnp.uint32),
            pltpu.VMEM((win, h), jnp.float32),
            pltpu.VMEM((win, h), jnp.float32),
            pltpu.VMEM((zr, h), jnp.float32),
            pltpu.VMEM_SHARED((n_pad, h), jnp.float32),
            pltpu.SemaphoreType.DMA,
            pltpu.SemaphoreType.DMA,
            pltpu.SemaphoreType.DMA,
            pltpu.SemaphoreType.DMA,
        ],
    )
    def k(m0_hbm, r0_hbm, m1_hbm, r1_hbm, out_hbm,
          iv0, iv1, mv0, mv1, mf0, mf1, zv, acc_sh, sm0, sm1, ss0, ss1):
        c = lax.axis_index("c")
        s = lax.axis_index("s")
        wid = s * NC + c
        ebase = wid * ept

        def unpack_into(mv, mf):
            def urow(r, c2):
                for j in range(hp // LANES):
                    u = mv[r, pl.ds(j * LANES, LANES)]
                    mf[r, pl.ds(j * LANES, LANES)] = jax.lax.bitcast_convert_type(
                        u << jnp.uint32(16), jnp.float32
                    )
                    mf[r, pl.ds(hp + j * LANES, LANES)] = jax.lax.bitcast_convert_type(
                        u & jnp.uint32(0xFFFF0000), jnp.float32
                    )
                return c2

            lax.fori_loop(0, win, urow, 0)

        def zrow(r, carry):
            for j in range(h // LANES):
                zv[r, pl.ds(j * LANES, LANES)] = jnp.zeros((LANES,), jnp.float32)
            return carry

        lax.fori_loop(0, zr, zrow, 0)
        for t in range(slab // zr):
            pltpu.sync_copy(zv, acc_sh.at[pl.ds(pl.multiple_of(s * slab + t * zr, 8), zr)])
        plsc.subcore_barrier()

        for m_hbm, row_hbm in ((m0_hbm, r0_hbm), (m1_hbm, r1_hbm)):

            def start_mload(w, mv, iv, sm):
                base = pl.multiple_of(ebase + w * win, 8)
                pltpu.async_copy(m_hbm.at[pl.ds(base, win)], mv, sm)
                pltpu.async_copy(row_hbm.at[pl.ds(base, win)], iv, sm)

            def wait_mload(mv, iv, sm):
                pltpu.make_async_copy(m_hbm.at[pl.ds(0, win)], mv, sm).wait()
                pltpu.make_async_copy(row_hbm.at[pl.ds(0, win)], iv, sm).wait()

            def start_scat(mf, iv, ss):
                pltpu.async_copy(mf, acc_sh.at[iv], ss, add=True)

            def wait_scat(mf, iv, ss):
                pltpu.make_async_copy(mf, acc_sh.at[iv], ss).wait()

            start_mload(0, mv0, iv0, sm0)
            start_mload(1, mv1, iv1, sm1)

            def body(t, carry):
                w = 2 * t
                wait_mload(mv0, iv0, sm0)
                unpack_into(mv0, mf0)
                start_scat(mf0, iv0, ss0)
                wait_mload(mv1, iv1, sm1)
                unpack_into(mv1, mf1)
                start_scat(mf1, iv1, ss1)

                @pl.when(w + 2 < nwin)
                def _():
                    wait_scat(mf0, iv0, ss0)
                    start_mload(w + 2, mv0, iv0, sm0)

                @pl.when(w + 3 < nwin)
                def _():
                    wait_scat(mf1, iv1, ss1)
                    start_mload(w + 3, mv1, iv1, sm1)

                return carry

            lax.fori_loop(0, npair, body, 0)
            wait_scat(mf0, iv0, ss0)
            wait_scat(mf1, iv1, ss1)

        plsc.subcore_barrier()
        sbase = pl.multiple_of(s * slab, 8)
        pltpu.sync_copy(
            acc_sh.at[pl.ds(sbase, slab)], out_hbm.at[c, pl.ds(sbase, slab)]
        )

    return k(m0, r0, m1, r1)


def _node_mlp_body(x_ref, p_ref, w3a_ref, w3b_ref, b3_ref, w4_ref, b4_ref, out_ref):
    agg = p_ref[0] + p_ref[1]
    xv = x_ref[...]
    hv = _silu(
        jnp.dot(xv, w3a_ref[...], preferred_element_type=jnp.float32)
        + jnp.dot(agg, w3b_ref[...], preferred_element_type=jnp.float32)
        + b3_ref[...]
    )
    out_ref[...] = (
        xv + jnp.dot(hv, w4_ref[...], preferred_element_type=jnp.float32) + b4_ref[...]
    )


def _node_mlp(x, partials, w3a, w3b, b3, w4, b4, bn=2000):
    n, d = x.shape
    h = w3a.shape[1]
    return pl.pallas_call(
        _node_mlp_body,
        grid=(n // bn,),
        in_specs=[
            pl.BlockSpec((bn, d), lambda i: (i, 0)),
            pl.BlockSpec((NC, bn, h), lambda i: (0, i, 0)),
            pl.BlockSpec((d, h), lambda i: (0, 0)),
            pl.BlockSpec((h, h), lambda i: (0, 0)),
            pl.BlockSpec((1, h), lambda i: (0, 0)),
            pl.BlockSpec((h, d), lambda i: (0, 0)),
            pl.BlockSpec((1, d), lambda i: (0, 0)),
        ],
        out_specs=pl.BlockSpec((bn, d), lambda i: (i, 0)),
        out_shape=jax.ShapeDtypeStruct((n, d), jnp.float32),
    )(x, partials, w3a, w3b, b3.reshape(1, h), w4, b4.reshape(1, d))


def kernel(x, edge_index, edge_attr, W1, b1, W2, b2, W3, b3, W4, b4):
    n, d = x.shape
    e = edge_index.shape[1]
    h = W2.shape[0]
    row = edge_index[0]
    col = edge_index[1]
    w1a, w1b, w1c = W1[:d], W1[d : 2 * d], W1[2 * d :].reshape(1, h)

    n_pad = 10240
    e_pad = 327680
    chunk = e_pad // 2
    npe = e_pad - e
    pad_i = jnp.arange(npe, dtype=jnp.int32)
    row_g = jnp.concatenate([row, (pad_i * 13) % n])
    col_g = jnp.concatenate([col, (pad_i * 29) % n])
    row_s = jnp.concatenate([row, n + pad_i % (n_pad - n)])
    attr_p = jnp.concatenate(
        [edge_attr.reshape(e), jnp.zeros((npe,), jnp.float32)]
    )

    xa, xb = _pre_tables(x, w1a, w1b, b1)
    pre0 = _gather_add(xa, xb, row_g[:chunk], col_g[:chunk], chunk)
    m0 = _edge_mlp(pre0, attr_p[:chunk].reshape(chunk // 1280, 10, 128), w1c, W2, b2)
    pre1 = _gather_add(xa, xb, row_g[chunk:], col_g[chunk:], chunk)
    m1 = _edge_mlp(pre1, attr_p[chunk:].reshape(chunk // 1280, 10, 128), w1c, W2, b2)
    partials = _scatter_partials(m0, row_s[:chunk], m1, row_s[chunk:], n_pad)
    out = _node_mlp(x, partials, W3[:d], W3[d:], b3, W4, b4)
    return out


# speedup vs baseline: 1.0340x; 1.0340x over previous
import functools

import jax
import jax.numpy as jnp
from jax import lax
from jax.experimental import pallas as pl
from jax.experimental.pallas import tpu as pltpu
from jax.experimental.pallas import tpu_sc as plsc

NC, NS, LANES = 2, 16, 16
NW = NC * NS


def _silu(v):
    return v * jax.nn.sigmoid(v)


def _pack_bf16_pairs(v):
    hh = v.shape[1] // 2
    lo = v[:, :hh].astype(jnp.bfloat16).astype(jnp.float32)
    hi = v[:, hh:].astype(jnp.bfloat16).astype(jnp.float32)
    lo_u = jax.lax.bitcast_convert_type(lo, jnp.uint32) >> 16
    hi_u = jax.lax.bitcast_convert_type(hi, jnp.uint32) & jnp.uint32(0xFFFF0000)
    return jax.lax.bitcast_convert_type(lo_u | hi_u, jnp.float32)


def _unpack_bf16_pairs(p):
    u = jax.lax.bitcast_convert_type(p, jnp.uint32)
    lo = jax.lax.bitcast_convert_type(u << 16, jnp.float32)
    hi = jax.lax.bitcast_convert_type(u & jnp.uint32(0xFFFF0000), jnp.float32)
    return jnp.concatenate([lo, hi], axis=1)


def _pre_tables_body(x_ref, w1a_ref, w1b_ref, b1_ref, xa_ref, xb_ref):
    x = x_ref[...]
    xa_ref[...] = (
        jnp.dot(x, w1a_ref[...], preferred_element_type=jnp.float32) + b1_ref[...]
    )
    xb_ref[...] = jnp.dot(x, w1b_ref[...], preferred_element_type=jnp.float32)


def _pre_tables(x, w1a, w1b, b1, bn=2000):
    n, d = x.shape
    h = w1a.shape[1]
    return pl.pallas_call(
        _pre_tables_body,
        grid=(n // bn,),
        in_specs=[
            pl.BlockSpec((bn, d), lambda i: (i, 0)),
            pl.BlockSpec((d, h), lambda i: (0, 0)),
            pl.BlockSpec((d, h), lambda i: (0, 0)),
            pl.BlockSpec((1, h), lambda i: (0, 0)),
        ],
        out_specs=[
            pl.BlockSpec((bn, h), lambda i: (i, 0)),
            pl.BlockSpec((bn, h), lambda i: (i, 0)),
        ],
        out_shape=[
            jax.ShapeDtypeStruct((n, h), jnp.float32),
            jax.ShapeDtypeStruct((n, h), jnp.float32),
        ],
    )(x, w1a, w1b, b1.reshape(1, h))


def _gather_add(xa, xb, row, col, e_total, win=128):
    n, h = xa.shape
    ew = e_total // NW
    nwin = ew // win
    npair = nwin // 2
    mesh = plsc.VectorSubcoreMesh(core_axis_name="c", subcore_axis_name="s")

    @functools.partial(
        pl.kernel,
        out_type=jax.ShapeDtypeStruct((e_total, h), jnp.float32),
        mesh=mesh,
        scratch_types=[
            pltpu.VMEM((win,), jnp.int32),
            pltpu.VMEM((win,), jnp.int32),
            pltpu.VMEM((win,), jnp.int32),
            pltpu.VMEM((win,), jnp.int32),
            pltpu.VMEM((win, h), jnp.float32),
            pltpu.VMEM((win, h), jnp.float32),
            pltpu.VMEM((win, h), jnp.float32),
            pltpu.VMEM((win, h), jnp.float32),
            pltpu.SemaphoreType.DMA,
            pltpu.SemaphoreType.DMA,
            pltpu.SemaphoreType.DMA,
            pltpu.SemaphoreType.DMA,
            pltpu.SemaphoreType.DMA,
            pltpu.SemaphoreType.DMA,
        ],
    )
    def k(xa_hbm, xb_hbm, row_hbm, col_hbm, pre_hbm,
          ra0, ca0, ra1, ca1, ga0, gb0, ga1, gb1,
          si0, si1, sg0, sg1, sw0, sw1):
        wid = lax.axis_index("s") * NC + lax.axis_index("c")
        ebase = wid * ew

        def start_idx(w, ra, ca, si):
            base = pl.multiple_of(ebase + w * win, 8)
            pltpu.async_copy(row_hbm.at[pl.ds(base, win)], ra, si)
            pltpu.async_copy(col_hbm.at[pl.ds(base, win)], ca, si)

        def wait_idx(ra, ca, si):
            pltpu.make_async_copy(row_hbm.at[pl.ds(0, win)], ra, si).wait()
            pltpu.make_async_copy(col_hbm.at[pl.ds(0, win)], ca, si).wait()

        def start_gather(ra, ca, ga, gb, sg):
            pltpu.async_copy(xa_hbm.at[ra], ga, sg)
            pltpu.async_copy(xb_hbm.at[ca], gb, sg)

        def wait_gather(ra, ca, ga, gb, sg):
            pltpu.make_async_copy(xa_hbm.at[ra], ga, sg).wait()
            pltpu.make_async_copy(xb_hbm.at[ca], gb, sg).wait()

        def start_wb(w, ga, sw):
            base = pl.multiple_of(ebase + w * win, 8)
            pltpu.async_copy(ga, pre_hbm.at[pl.ds(base, win)], sw)

        def wait_wb(ga, sw):
            pltpu.make_async_copy(ga, pre_hbm.at[pl.ds(0, win)], sw).wait()

        def add_into(ga, gb):
            def add_row(r, c2):
                for j in range(h // LANES):
                    sl = pl.ds(j * LANES, LANES)
                    ga[r, sl] = ga[r, sl] + gb[r, sl]
                return c2

            lax.fori_loop(0, win, add_row, 0)

        start_idx(0, ra0, ca0, si0)
        start_idx(1, ra1, ca1, si1)
        wait_idx(ra0, ca0, si0)
        start_gather(ra0, ca0, ga0, gb0, sg0)

        def body(t, carry):
            w = 2 * t
            wait_gather(ra0, ca0, ga0, gb0, sg0)

            @pl.when(w + 2 < nwin)
            def _():
                start_idx(w + 2, ra0, ca0, si0)

            @pl.when(t > 0)
            def _():
                wait_wb(ga1, sw1)

            wait_idx(ra1, ca1, si1)
            start_gather(ra1, ca1, ga1, gb1, sg1)
            add_into(ga0, gb0)
            start_wb(w, ga0, sw0)

            wait_gather(ra1, ca1, ga1, gb1, sg1)

            @pl.when(w + 3 < nwin)
            def _():
                start_idx(w + 3, ra1, ca1, si1)

            @pl.when(w + 2 < nwin)
            def _():
                wait_wb(ga0, sw0)
                wait_idx(ra0, ca0, si0)
                start_gather(ra0, ca0, ga0, gb0, sg0)

            add_into(ga1, gb1)
            start_wb(w + 1, ga1, sw1)
            return carry

        lax.fori_loop(0, npair, body, 0)
        wait_wb(ga0, sw0)
        wait_wb(ga1, sw1)

    return k(xa, xb, row, col)


def _edge_mlp_body(nsub, pre_ref, attr_ref, w1c_ref, w2_ref, b2_ref, out_ref):
    w1c = w1c_ref[...]
    w2 = w2_ref[...]
    b2 = b2_ref[...]
    for t in range(nsub):
        a = attr_ref[0, pl.ds(t, 1), :]
        c = lax.dot_general(
            a, w1c, (((0,), (0,)), ((), ())), preferred_element_type=jnp.float32
        )
        p = pre_ref[pl.ds(t * 128, 128), :]
        h1 = _silu(p + c)
        m = _silu(jnp.dot(h1, w2, preferred_element_type=jnp.float32) + b2)
        out_ref[pl.ds(t * 128, 128), :] = m


def _edge_mlp(pre, attr3, w1c, w2, b2, bt=1280):
    e, h = pre.shape
    nsub = bt // 128
    return pl.pallas_call(
        functools.partial(_edge_mlp_body, nsub),
        grid=(e // bt,),
        in_specs=[
            pl.BlockSpec((bt, h), lambda i: (i, 0)),
            pl.BlockSpec((1, nsub, 128), lambda i: (i, 0, 0)),
            pl.BlockSpec((1, h), lambda i: (0, 0)),
            pl.BlockSpec((h, h), lambda i: (0, 0)),
            pl.BlockSpec((1, h), lambda i: (0, 0)),
        ],
        out_specs=pl.BlockSpec((bt, h), lambda i: (i, 0)),
        out_shape=jax.ShapeDtypeStruct((e, h), jnp.float32),
    )(pre, attr3, w1c, w2, b2.reshape(1, h))


def _scatter_partials(m0, r0, m1, r1, n_pad, win=128, zr=64):
    e_chunk, h = m0.shape
    ept = e_chunk // NW
    nwin = ept // win
    npair = nwin // 2
    slab = n_pad // NS
    mesh = plsc.VectorSubcoreMesh(core_axis_name="c", subcore_axis_name="s")

    @functools.partial(
        pl.kernel,
        out_type=jax.ShapeDtypeStruct((NC, n_pad, h), jnp.float32),
        mesh=mesh,
        scratch_types=[
            pltpu.VMEM((win,), jnp.int32),
            pltpu.VMEM((win,), jnp.int32),
            pltpu.VMEM((win, h), jnp.float32),
            pltpu.VMEM((win, h), jnp.float32),
            pltpu.VMEM((zr, h), jnp.float32),
            pltpu.VMEM_SHARED((n_pad, h), jnp.float32),
            pltpu.SemaphoreType.DMA,
            pltpu.SemaphoreType.DMA,
            pltpu.SemaphoreType.DMA,
            pltpu.SemaphoreType.DMA,
        ],
    )
    def k(m0_hbm, r0_hbm, m1_hbm, r1_hbm, out_hbm,
          iv0, iv1, mv0, mv1, zv, acc_sh, sm0, sm1, ss0, ss1):
        c = lax.axis_index("c")
        s = lax.axis_index("s")
        wid = s * NC + c
        ebase = wid * ept

        def zrow(r, carry):
            for j in range(h // LANES):
                zv[r, pl.ds(j * LANES, LANES)] = jnp.zeros((LANES,), jnp.float32)
            return carry

        lax.fori_loop(0, zr, zrow, 0)
        for t in range(slab // zr):
            pltpu.sync_copy(zv, acc_sh.at[pl.ds(pl.multiple_of(s * slab + t * zr, 8), zr)])
        plsc.subcore_barrier()

        for m_hbm, row_hbm in ((m0_hbm, r0_hbm), (m1_hbm, r1_hbm)):

            def start_mload(w, mv, iv, sm):
                base = pl.multiple_of(ebase + w * win, 8)
                pltpu.async_copy(m_hbm.at[pl.ds(base, win)], mv, sm)
                pltpu.async_copy(row_hbm.at[pl.ds(base, win)], iv, sm)

            def wait_mload(mv, iv, sm):
                pltpu.make_async_copy(m_hbm.at[pl.ds(0, win)], mv, sm).wait()
                pltpu.make_async_copy(row_hbm.at[pl.ds(0, win)], iv, sm).wait()

            def start_scat(mv, iv, ss):
                pltpu.async_copy(mv, acc_sh.at[iv], ss, add=True)

            def wait_scat(mv, iv, ss):
                pltpu.make_async_copy(mv, acc_sh.at[iv], ss).wait()

            start_mload(0, mv0, iv0, sm0)
            start_mload(1, mv1, iv1, sm1)

            def body(t, carry):
                w = 2 * t
                wait_mload(mv0, iv0, sm0)
                start_scat(mv0, iv0, ss0)
                wait_mload(mv1, iv1, sm1)
                start_scat(mv1, iv1, ss1)

                @pl.when(w + 2 < nwin)
                def _():
                    wait_scat(mv0, iv0, ss0)
                    start_mload(w + 2, mv0, iv0, sm0)

                @pl.when(w + 3 < nwin)
                def _():
                    wait_scat(mv1, iv1, ss1)
                    start_mload(w + 3, mv1, iv1, sm1)

                return carry

            lax.fori_loop(0, npair, body, 0)
            wait_scat(mv0, iv0, ss0)
            wait_scat(mv1, iv1, ss1)

        plsc.subcore_barrier()
        sbase = pl.multiple_of(s * slab, 8)
        pltpu.sync_copy(
            acc_sh.at[pl.ds(sbase, slab)], out_hbm.at[c, pl.ds(sbase, slab)]
        )

    return k(m0, r0, m1, r1)


def _node_mlp_body(x_ref, p_ref, w3a_ref, w3b_ref, b3_ref, w4_ref, b4_ref, out_ref):
    agg = p_ref[0] + p_ref[1]
    xv = x_ref[...]
    hv = _silu(
        jnp.dot(xv, w3a_ref[...], preferred_element_type=jnp.float32)
        + jnp.dot(agg, w3b_ref[...], preferred_element_type=jnp.float32)
        + b3_ref[...]
    )
    out_ref[...] = (
        xv + jnp.dot(hv, w4_ref[...], preferred_element_type=jnp.float32) + b4_ref[...]
    )


def _node_mlp(x, partials, w3a, w3b, b3, w4, b4, bn=2000):
    n, d = x.shape
    h = w3a.shape[1]
    return pl.pallas_call(
        _node_mlp_body,
        grid=(n // bn,),
        in_specs=[
            pl.BlockSpec((bn, d), lambda i: (i, 0)),
            pl.BlockSpec((NC, bn, h), lambda i: (0, i, 0)),
            pl.BlockSpec((d, h), lambda i: (0, 0)),
            pl.BlockSpec((h, h), lambda i: (0, 0)),
            pl.BlockSpec((1, h), lambda i: (0, 0)),
            pl.BlockSpec((h, d), lambda i: (0, 0)),
            pl.BlockSpec((1, d), lambda i: (0, 0)),
        ],
        out_specs=pl.BlockSpec((bn, d), lambda i: (i, 0)),
        out_shape=jax.ShapeDtypeStruct((n, d), jnp.float32),
    )(x, partials, w3a, w3b, b3.reshape(1, h), w4, b4.reshape(1, d))


def kernel(x, edge_index, edge_attr, W1, b1, W2, b2, W3, b3, W4, b4):
    n, d = x.shape
    e = edge_index.shape[1]
    h = W2.shape[0]
    row = edge_index[0]
    col = edge_index[1]
    w1a, w1b, w1c = W1[:d], W1[d : 2 * d], W1[2 * d :].reshape(1, h)

    n_pad = 10240
    e_pad = 327680
    chunk = e_pad // 2
    npe = e_pad - e
    pad_i = jnp.arange(npe, dtype=jnp.int32)
    row_g = jnp.concatenate([row, (pad_i * 13) % n])
    col_g = jnp.concatenate([col, (pad_i * 29) % n])
    row_s = jnp.concatenate([row, n + pad_i % (n_pad - n)])
    attr_p = jnp.concatenate(
        [edge_attr.reshape(e), jnp.zeros((npe,), jnp.float32)]
    )

    xa, xb = _pre_tables(x, w1a, w1b, b1)
    pre0 = _gather_add(xa, xb, row_g[:chunk], col_g[:chunk], chunk)
    m0 = _edge_mlp(pre0, attr_p[:chunk].reshape(chunk // 1280, 10, 128), w1c, W2, b2)
    pre1 = _gather_add(xa, xb, row_g[chunk:], col_g[chunk:], chunk)
    m1 = _edge_mlp(pre1, attr_p[chunk:].reshape(chunk // 1280, 10, 128), w1c, W2, b2)
    partials = _scatter_partials(m0, row_s[:chunk], m1, row_s[chunk:], n_pad)
    out = _node_mlp(x, partials, W3[:d], W3[d:], b3, W4, b4)
    return out


# speedup vs baseline: 1.0362x; 1.0021x over previous
import functools

import jax
import jax.numpy as jnp
from jax import lax
from jax.experimental import pallas as pl
from jax.experimental.pallas import tpu as pltpu
from jax.experimental.pallas import tpu_sc as plsc

NC, NS, LANES = 2, 16, 16
NW = NC * NS


def _silu(v):
    return v * jax.nn.sigmoid(v)


def _pre_tables_body(x_ref, w1a_ref, w1b_ref, b1_ref, xa_ref, xb_ref):
    x = x_ref[...]
    xa_ref[...] = (
        jnp.dot(x, w1a_ref[...], preferred_element_type=jnp.float32) + b1_ref[...]
    )
    xb_ref[...] = jnp.dot(x, w1b_ref[...], preferred_element_type=jnp.float32)


def _pre_tables(x, w1a, w1b, b1, bn=2000):
    n, d = x.shape
    h = w1a.shape[1]
    return pl.pallas_call(
        _pre_tables_body,
        grid=(n // bn,),
        in_specs=[
            pl.BlockSpec((bn, d), lambda i: (i, 0)),
            pl.BlockSpec((d, h), lambda i: (0, 0)),
            pl.BlockSpec((d, h), lambda i: (0, 0)),
            pl.BlockSpec((1, h), lambda i: (0, 0)),
        ],
        out_specs=[
            pl.BlockSpec((bn, h), lambda i: (i, 0)),
            pl.BlockSpec((bn, h), lambda i: (i, 0)),
        ],
        out_shape=[
            jax.ShapeDtypeStruct((n, h), jnp.float32),
            jax.ShapeDtypeStruct((n, h), jnp.float32),
        ],
    )(x, w1a, w1b, b1.reshape(1, h))


def _gather_add(xa, xb, row, col, e_total, win=128):
    n, h = xa.shape
    ew = e_total // NW
    nwin = ew // win
    npair = nwin // 2
    mesh = plsc.VectorSubcoreMesh(core_axis_name="c", subcore_axis_name="s")

    @functools.partial(
        pl.kernel,
        out_type=jax.ShapeDtypeStruct((e_total, h), jnp.float32),
        mesh=mesh,
        scratch_types=[
            pltpu.VMEM((win,), jnp.int32),
            pltpu.VMEM((win,), jnp.int32),
            pltpu.VMEM((win,), jnp.int32),
            pltpu.VMEM((win,), jnp.int32),
            pltpu.VMEM((win, h), jnp.float32),
            pltpu.VMEM((win, h), jnp.float32),
            pltpu.VMEM((win, h), jnp.float32),
            pltpu.VMEM((win, h), jnp.float32),
            pltpu.SemaphoreType.DMA,
            pltpu.SemaphoreType.DMA,
            pltpu.SemaphoreType.DMA,
            pltpu.SemaphoreType.DMA,
            pltpu.SemaphoreType.DMA,
            pltpu.SemaphoreType.DMA,
        ],
    )
    def k(xa_hbm, xb_hbm, row_hbm, col_hbm, pre_hbm,
          ra0, ca0, ra1, ca1, ga0, gb0, ga1, gb1,
          si0, si1, sg0, sg1, sw0, sw1):
        wid = lax.axis_index("s") * NC + lax.axis_index("c")
        ebase = wid * ew

        def start_idx(w, ra, ca, si):
            base = pl.multiple_of(ebase + w * win, 8)
            pltpu.async_copy(row_hbm.at[pl.ds(base, win)], ra, si)
            pltpu.async_copy(col_hbm.at[pl.ds(base, win)], ca, si)

        def wait_idx(ra, ca, si):
            pltpu.make_async_copy(row_hbm.at[pl.ds(0, win)], ra, si).wait()
            pltpu.make_async_copy(col_hbm.at[pl.ds(0, win)], ca, si).wait()

        def start_gather(ra, ca, ga, gb, sg):
            pltpu.async_copy(xa_hbm.at[ra], ga, sg)
            pltpu.async_copy(xb_hbm.at[ca], gb, sg)

        def wait_gather(ra, ca, ga, gb, sg):
            pltpu.make_async_copy(xa_hbm.at[ra], ga, sg).wait()
            pltpu.make_async_copy(xb_hbm.at[ca], gb, sg).wait()

        def start_wb(w, ga, sw):
            base = pl.multiple_of(ebase + w * win, 8)
            pltpu.async_copy(ga, pre_hbm.at[pl.ds(base, win)], sw)

        def wait_wb(ga, sw):
            pltpu.make_async_copy(ga, pre_hbm.at[pl.ds(0, win)], sw).wait()

        def add_into(ga, gb):
            def add_row(r, c2):
                for j in range(h // LANES):
                    sl = pl.ds(j * LANES, LANES)
                    ga[r, sl] = ga[r, sl] + gb[r, sl]
                return c2

            lax.fori_loop(0, win, add_row, 0)

        start_idx(0, ra0, ca0, si0)
        start_idx(1, ra1, ca1, si1)
        wait_idx(ra0, ca0, si0)
        start_gather(ra0, ca0, ga0, gb0, sg0)

        def body(t, carry):
            w = 2 * t
            wait_gather(ra0, ca0, ga0, gb0, sg0)

            @pl.when(w + 2 < nwin)
            def _():
                start_idx(w + 2, ra0, ca0, si0)

            @pl.when(t > 0)
            def _():
                wait_wb(ga1, sw1)

            wait_idx(ra1, ca1, si1)
            start_gather(ra1, ca1, ga1, gb1, sg1)
            add_into(ga0, gb0)
            start_wb(w, ga0, sw0)

            wait_gather(ra1, ca1, ga1, gb1, sg1)

            @pl.when(w + 3 < nwin)
            def _():
                start_idx(w + 3, ra1, ca1, si1)

            @pl.when(w + 2 < nwin)
            def _():
                wait_wb(ga0, sw0)
                wait_idx(ra0, ca0, si0)
                start_gather(ra0, ca0, ga0, gb0, sg0)

            add_into(ga1, gb1)
            start_wb(w + 1, ga1, sw1)
            return carry

        lax.fori_loop(0, npair, body, 0)
        wait_wb(ga0, sw0)
        wait_wb(ga1, sw1)

    return k(xa, xb, row, col)


def _edge_mlp_body(nsub, pre_ref, attr_ref, w1c_ref, w2_ref, b2_ref, out_ref):
    w1c = w1c_ref[...]
    w2 = w2_ref[...]
    b2 = b2_ref[...]
    for t in range(nsub):
        a = attr_ref[0, pl.ds(t, 1), :]
        c = lax.dot_general(
            a, w1c, (((0,), (0,)), ((), ())), preferred_element_type=jnp.float32
        )
        p = pre_ref[pl.ds(t * 128, 128), :]
        h1 = _silu(p + c)
        m = _silu(jnp.dot(h1, w2, preferred_element_type=jnp.float32) + b2)
        out_ref[pl.ds(t * 128, 128), :] = m


def _edge_mlp(pre, attr3, w1c, w2, b2, bt=1280):
    e, h = pre.shape
    nsub = bt // 128
    return pl.pallas_call(
        functools.partial(_edge_mlp_body, nsub),
        grid=(e // bt,),
        in_specs=[
            pl.BlockSpec((bt, h), lambda i: (i, 0)),
            pl.BlockSpec((1, nsub, 128), lambda i: (i, 0, 0)),
            pl.BlockSpec((1, h), lambda i: (0, 0)),
            pl.BlockSpec((h, h), lambda i: (0, 0)),
            pl.BlockSpec((1, h), lambda i: (0, 0)),
        ],
        out_specs=pl.BlockSpec((bt, h), lambda i: (i, 0)),
        out_shape=jax.ShapeDtypeStruct((e, h), jnp.float32),
    )(pre, attr3, w1c, w2, b2.reshape(1, h))


def _scatter_partials(m0, r0, m1, r1, n_pad, win=128, zr=64):
    e_chunk, h = m0.shape
    ept = e_chunk // NW
    nwin = ept // win
    npair = nwin // 2
    slab = n_pad // NS
    mesh = plsc.VectorSubcoreMesh(core_axis_name="c", subcore_axis_name="s")

    @functools.partial(
        pl.kernel,
        out_type=jax.ShapeDtypeStruct((NC, n_pad, h), jnp.float32),
        mesh=mesh,
        scratch_types=[
            pltpu.VMEM((win,), jnp.int32),
            pltpu.VMEM((win,), jnp.int32),
            pltpu.VMEM((win, h), jnp.float32),
            pltpu.VMEM((win, h), jnp.float32),
            pltpu.VMEM((zr, h), jnp.float32),
            pltpu.VMEM_SHARED((n_pad, h), jnp.float32),
            pltpu.SemaphoreType.DMA,
            pltpu.SemaphoreType.DMA,
            pltpu.SemaphoreType.DMA,
            pltpu.SemaphoreType.DMA,
        ],
    )
    def k(m0_hbm, r0_hbm, m1_hbm, r1_hbm, out_hbm,
          iv0, iv1, mv0, mv1, zv, acc_sh, sm0, sm1, ss0, ss1):
        c = lax.axis_index("c")
        s = lax.axis_index("s")
        wid = s * NC + c
        ebase = wid * ept

        def zrow(r, carry):
            for j in range(h // LANES):
                zv[r, pl.ds(j * LANES, LANES)] = jnp.zeros((LANES,), jnp.float32)
            return carry

        lax.fori_loop(0, zr, zrow, 0)
        for t in range(slab // zr):
            pltpu.sync_copy(zv, acc_sh.at[pl.ds(pl.multiple_of(s * slab + t * zr, 8), zr)])
        plsc.subcore_barrier()

        for m_hbm, row_hbm in ((m0_hbm, r0_hbm), (m1_hbm, r1_hbm)):

            def start_mload(w, mv, iv, sm):
                base = pl.multiple_of(ebase + w * win, 8)
                pltpu.async_copy(m_hbm.at[pl.ds(base, win)], mv, sm)
                pltpu.async_copy(row_hbm.at[pl.ds(base, win)], iv, sm)

            def wait_mload(mv, iv, sm):
                pltpu.make_async_copy(m_hbm.at[pl.ds(0, win)], mv, sm).wait()
                pltpu.make_async_copy(row_hbm.at[pl.ds(0, win)], iv, sm).wait()

            def start_scat(mv, iv, ss):
                pltpu.async_copy(mv, acc_sh.at[iv], ss, add=True)

            def wait_scat(mv, iv, ss):
                pltpu.make_async_copy(mv, acc_sh.at[iv], ss).wait()

            start_mload(0, mv0, iv0, sm0)
            start_mload(1, mv1, iv1, sm1)

            def body(t, carry):
                w = 2 * t
                wait_mload(mv0, iv0, sm0)
                start_scat(mv0, iv0, ss0)
                wait_mload(mv1, iv1, sm1)
                start_scat(mv1, iv1, ss1)

                @pl.when(w + 2 < nwin)
                def _():
                    wait_scat(mv0, iv0, ss0)
                    start_mload(w + 2, mv0, iv0, sm0)

                @pl.when(w + 3 < nwin)
                def _():
                    wait_scat(mv1, iv1, ss1)
                    start_mload(w + 3, mv1, iv1, sm1)

                return carry

            lax.fori_loop(0, npair, body, 0)
            wait_scat(mv0, iv0, ss0)
            wait_scat(mv1, iv1, ss1)

        plsc.subcore_barrier()
        sbase = pl.multiple_of(s * slab, 8)
        pltpu.sync_copy(
            acc_sh.at[pl.ds(sbase, slab)], out_hbm.at[c, pl.ds(sbase, slab)]
        )

    return k(m0, r0, m1, r1)


def _node_mlp_body(x_ref, p_ref, w3a_ref, w3b_ref, b3_ref, w4_ref, b4_ref, out_ref):
    agg = p_ref[0] + p_ref[1]
    xv = x_ref[...]
    hv = _silu(
        jnp.dot(xv, w3a_ref[...], preferred_element_type=jnp.float32)
        + jnp.dot(agg, w3b_ref[...], preferred_element_type=jnp.float32)
        + b3_ref[...]
    )
    out_ref[...] = (
        xv + jnp.dot(hv, w4_ref[...], preferred_element_type=jnp.float32) + b4_ref[...]
    )


def _node_mlp(x, partials, w3a, w3b, b3, w4, b4, bn=2000):
    n, d = x.shape
    h = w3a.shape[1]
    return pl.pallas_call(
        _node_mlp_body,
        grid=(n // bn,),
        in_specs=[
            pl.BlockSpec((bn, d), lambda i: (i, 0)),
            pl.BlockSpec((NC, bn, h), lambda i: (0, i, 0)),
            pl.BlockSpec((d, h), lambda i: (0, 0)),
            pl.BlockSpec((h, h), lambda i: (0, 0)),
            pl.BlockSpec((1, h), lambda i: (0, 0)),
            pl.BlockSpec((h, d), lambda i: (0, 0)),
            pl.BlockSpec((1, d), lambda i: (0, 0)),
        ],
        out_specs=pl.BlockSpec((bn, d), lambda i: (i, 0)),
        out_shape=jax.ShapeDtypeStruct((n, d), jnp.float32),
    )(x, partials, w3a, w3b, b3.reshape(1, h), w4, b4.reshape(1, d))


def kernel(x, edge_index, edge_attr, W1, b1, W2, b2, W3, b3, W4, b4):
    n, d = x.shape
    e = edge_index.shape[1]
    h = W2.shape[0]
    row = edge_index[0]
    col = edge_index[1]
    w1a, w1b, w1c = W1[:d], W1[d : 2 * d], W1[2 * d :].reshape(1, h)

    n_pad = 10240
    e_pad = 327680
    chunk = e_pad // 2
    npe = e_pad - e
    pad_i = jnp.arange(npe, dtype=jnp.int32)
    row_g = jnp.concatenate([row, (pad_i * 13) % n])
    col_g = jnp.concatenate([col, (pad_i * 29) % n])
    row_s = jnp.concatenate([row, n + pad_i % (n_pad - n)])
    attr_p = jnp.concatenate(
        [edge_attr.reshape(e), jnp.zeros((npe,), jnp.float32)]
    )

    xa, xb = _pre_tables(x, w1a, w1b, b1)
    pre0 = _gather_add(xa, xb, row_g[:chunk], col_g[:chunk], chunk)
    m0 = _edge_mlp(pre0, attr_p[:chunk].reshape(chunk // 1280, 10, 128), w1c, W2, b2)
    pre1 = _gather_add(xa, xb, row_g[chunk:], col_g[chunk:], chunk)
    m1 = _edge_mlp(pre1, attr_p[chunk:].reshape(chunk // 1280, 10, 128), w1c, W2, b2)
    partials = _scatter_partials(m0, row_s[:chunk], m1, row_s[chunk:], n_pad)
    out = _node_mlp(x, partials, W3[:d], W3[d:], b3, W4, b4)
    return out
